# baseline (speedup 1.0000x reference)
import functools

import jax
import jax.numpy as jnp
from jax import lax
from jax.experimental import pallas as pl
from jax.experimental.pallas import tpu as pltpu
from jax.experimental.pallas import tpu_sc as plsc

VOCAB = 1000000
DIM = 64
B = 4096
L = 200
N_CLASSES = 2

_INFO = plsc.get_sparse_core_info()
NC = _INFO.num_cores
NS = _INFO.num_subcores
NW = NC * NS
S_PER_W = B // NW
INV_L = 1.0 / L

BLKN = 65536
NBLK = (VOCAB + BLKN - 1) // BLKN
NPAD = 8


def _proj_body(w_ref, t_ref, o_ref):
    o_ref[...] = jnp.dot(
        w_ref[...], t_ref[...], preferred_element_type=jnp.float32)


_proj = pl.pallas_call(
    _proj_body,
    grid=(NBLK,),
    in_specs=[
        pl.BlockSpec((NPAD, DIM), lambda i: (0, 0)),
        pl.BlockSpec((DIM, BLKN), lambda i: (0, i)),
    ],
    out_specs=pl.BlockSpec((NPAD, BLKN), lambda i: (0, i)),
    out_shape=jax.ShapeDtypeStruct((NPAD, VOCAB), jnp.float32),
)

_sc_mesh = plsc.VectorSubcoreMesh(core_axis_name="c", subcore_axis_name="s")


@functools.partial(
    pl.kernel,
    mesh=_sc_mesh,
    compiler_params=pltpu.CompilerParams(
        use_tc_tiling_on_sc=False, needs_layout_passes=False),
    out_type=jax.ShapeDtypeStruct((B * N_CLASSES,), jnp.float32),
    scratch_types=[
        pltpu.VMEM((L, S_PER_W), jnp.int32),
        pltpu.VMEM((2, (L // 8) * S_PER_W, NPAD), jnp.float32),
        pltpu.VMEM((N_CLASSES * S_PER_W,), jnp.float32),
        pltpu.VMEM((16,), jnp.float32),
        pltpu.SemaphoreType.DMA,
    ],
)
def _sc_pool(xt_hbm, p_hbm, b_hbm, out_hbm, idx_v, g_v, out_v, b_v, sem0):
    wid = lax.axis_index("s") * NC + lax.axis_index("c")
    base = wid * S_PER_W
    pltpu.sync_copy(xt_hbm.at[:, pl.ds(base, S_PER_W)], idx_v)
    pltpu.sync_copy(b_hbm, b_v)

    NCHUNK = 8
    TCHUNK = L // NCHUNK
    CROWS = TCHUNK * S_PER_W

    def fire(c):
        buf = g_v.at[c % 2]

        def fire_t(tl, carry):
            t = c * TCHUNK + tl
            pltpu.async_copy(
                p_hbm.at[idx_v.at[t]],
                buf.at[pl.ds(tl * S_PER_W, S_PER_W)], sem0)
            return carry
        lax.fori_loop(0, TCHUNK, fire_t, 0)

    def drain(c):
        pltpu.make_async_copy(
            p_hbm.at[pl.ds(0, CROWS)], g_v.at[c % 2], sem0).wait()

    zero = jnp.zeros((16,), jnp.float32)
    ngrp = S_PER_W * N_CLASSES // 16
    iota16 = lax.iota(jnp.int32, 16)
    colv = iota16 & 1
    half = iota16 >> 1
    rowconst = [j * 8 + half for j in range(ngrp)]

    def accum(c, accs):
        buf = g_v.at[c % 2]

        def tok_body(tl, accs):
            rowbase = tl * S_PER_W
            new = []
            for j in range(ngrp):
                v = plsc.load_gather(buf, [rowconst[j] + rowbase, colv])
                new.append(accs[j] + v)
            return tuple(new)
        return lax.fori_loop(0, TCHUNK, tok_body, accs)

    fire(0)
    accs = (zero,) * ngrp
    for c in range(NCHUNK):
        if c + 1 < NCHUNK:
            fire(c + 1)
        drain(c)
        accs = accum(c, accs)

    b_il = b_v[pl.ds(0, 16)]
    for j in range(ngrp):
        out_v[pl.ds(j * 16, 16)] = accs[j] * INV_L + b_il
    pltpu.sync_copy(out_v,
                    out_hbm.at[pl.ds(base * N_CLASSES,
                                     S_PER_W * N_CLASSES)])


@jax.jit
def kernel(x, table, W, b):
    tableT = table.T
    w8 = jnp.zeros((NPAD, DIM), jnp.float32).at[:N_CLASSES].set(W)
    proj = _proj(w8, tableT)
    xt = x.T.astype(jnp.int32)
    b_il = jnp.tile(b, 16 // N_CLASSES)
    out_flat = _sc_pool(xt, proj.T, b_il)
    return out_flat.reshape(B, N_CLASSES)


# speedup vs baseline: 5.1746x; 5.1746x over previous
import functools

import jax
import jax.numpy as jnp
from jax import lax
from jax.experimental import pallas as pl
from jax.experimental.pallas import tpu as pltpu
from jax.experimental.pallas import tpu_sc as plsc

VOCAB = 1000000
DIM = 64
B = 4096
L = 200
N_CLASSES = 2

_INFO = plsc.get_sparse_core_info()
NC = _INFO.num_cores
NS = _INFO.num_subcores
NW = NC * NS
S_PER_W = B // NW
INV_L = 1.0 / L

BLKN = 65536
NBLK = (VOCAB + BLKN - 1) // BLKN


def _proj_body(w_ref, t_ref, o_ref):
    p = jnp.dot(w_ref[...], t_ref[...], preferred_element_type=jnp.float32)
    u0 = lax.bitcast_convert_type(
        p[0].astype(jnp.bfloat16), jnp.uint16).astype(jnp.uint32)
    u1 = lax.bitcast_convert_type(
        p[1].astype(jnp.bfloat16), jnp.uint16).astype(jnp.uint32)
    o_ref[...] = lax.bitcast_convert_type((u0 << 16) | u1, jnp.float32)


_proj = pl.pallas_call(
    _proj_body,
    grid=(NBLK,),
    in_specs=[
        pl.BlockSpec((N_CLASSES, DIM), lambda i: (0, 0)),
        pl.BlockSpec((DIM, BLKN), lambda i: (0, i)),
    ],
    out_specs=pl.BlockSpec((BLKN,), lambda i: (i,)),
    out_shape=jax.ShapeDtypeStruct((VOCAB,), jnp.float32),
)

_sc_mesh = plsc.VectorSubcoreMesh(core_axis_name="c", subcore_axis_name="s")


@functools.partial(
    pl.kernel,
    mesh=_sc_mesh,
    compiler_params=pltpu.CompilerParams(
        use_tc_tiling_on_sc=False, needs_layout_passes=False),
    out_type=jax.ShapeDtypeStruct((N_CLASSES * B,), jnp.float32),
    scratch_types=[
        pltpu.VMEM((L, S_PER_W), jnp.int32),
        pltpu.VMEM((2, (L // 8) * S_PER_W), jnp.float32),
        pltpu.VMEM((N_CLASSES * S_PER_W,), jnp.float32),
        pltpu.VMEM((16,), jnp.float32),
        pltpu.SemaphoreType.DMA,
    ],
)
def _sc_pool(xt_hbm, pf_hbm, b_hbm, out_hbm, idx_v, g_v, out_v, b_v, sem0):
    wid = lax.axis_index("s") * NC + lax.axis_index("c")
    base = wid * S_PER_W
    pltpu.sync_copy(xt_hbm.at[:, pl.ds(base, S_PER_W)], idx_v)
    pltpu.sync_copy(b_hbm, b_v)

    NCHUNK = 8
    TCHUNK = L // NCHUNK
    CROWS = TCHUNK * S_PER_W

    def fire(c):
        buf = g_v.at[c % 2]

        def fire_t(tl, carry):
            t = c * TCHUNK + tl
            pltpu.async_copy(
                pf_hbm.at[idx_v.at[t]],
                buf.at[pl.ds(tl * S_PER_W, S_PER_W)], sem0)
            return carry
        lax.fori_loop(0, TCHUNK, fire_t, 0)

    def drain(c):
        pltpu.make_async_copy(
            pf_hbm.at[pl.ds(0, CROWS)], g_v.at[c % 2], sem0).wait()

    zero = jnp.zeros((16,), jnp.float32)
    ngrp = S_PER_W // 16
    himask = jnp.full((16,), 0xFFFF0000, jnp.uint32)

    def accum(c, accs):
        buf = g_v.at[c % 2]

        def tok_body(tl, accs):
            goff = tl * S_PER_W
            new = list(accs)
            for j in range(ngrp):
                w = buf[pl.ds(goff + j * 16, 16)]
                u = plsc.bitcast(w, jnp.uint32)
                c0 = plsc.bitcast(u & himask, jnp.float32)
                c1 = plsc.bitcast(u << 16, jnp.float32)
                new[j] = new[j] + c0
                new[ngrp + j] = new[ngrp + j] + c1
            return tuple(new)
        return lax.fori_loop(0, TCHUNK, tok_body, accs)

    fire(0)
    accs = (zero,) * (2 * ngrp)
    for c in range(NCHUNK):
        if c + 1 < NCHUNK:
            fire(c + 1)
        drain(c)
        accs = accum(c, accs)

    bvec = b_v[pl.ds(0, 16)]
    b0 = bvec[0]
    b1 = bvec[1]
    for j in range(ngrp):
        out_v[pl.ds(j * 16, 16)] = accs[j] * INV_L + b0
        out_v[pl.ds(S_PER_W + j * 16, 16)] = accs[ngrp + j] * INV_L + b1
    pltpu.sync_copy(out_v.at[pl.ds(0, S_PER_W)],
                    out_hbm.at[pl.ds(base, S_PER_W)])
    pltpu.sync_copy(out_v.at[pl.ds(S_PER_W, S_PER_W)],
                    out_hbm.at[pl.ds(B + base, S_PER_W)])


@jax.jit
def kernel(x, table, W, b):
    tableT = table.T
    pf = _proj(W, tableT)
    xt = x.T.astype(jnp.int32)
    bpad = jnp.zeros((16,), jnp.float32).at[:N_CLASSES].set(b)
    out_t = _sc_pool(xt, pf, bpad)
    return out_t.reshape(N_CLASSES, B).T


# speedup vs baseline: 5.2662x; 1.0177x over previous
import functools

import jax
import jax.numpy as jnp
from jax import lax
from jax.experimental import pallas as pl
from jax.experimental.pallas import tpu as pltpu
from jax.experimental.pallas import tpu_sc as plsc

VOCAB = 1000000
DIM = 64
B = 4096
L = 200
N_CLASSES = 2

_INFO = plsc.get_sparse_core_info()
NC = _INFO.num_cores
NS = _INFO.num_subcores
NW = NC * NS
S_PER_W = B // NW
INV_L = 1.0 / L

BLKN = 65536
NBLK = (VOCAB + BLKN - 1) // BLKN
IDX_PER_W = L * S_PER_W
WPB = NW // NBLK


def _proj_body(w_ref, t_ref, x_ref, o_ref, xw_ref):
    p = jnp.dot(w_ref[...], t_ref[...], preferred_element_type=jnp.float32)
    u0 = lax.bitcast_convert_type(
        p[0].astype(jnp.bfloat16), jnp.uint16).astype(jnp.uint32)
    u1 = lax.bitcast_convert_type(
        p[1].astype(jnp.bfloat16), jnp.uint16).astype(jnp.uint32)
    o_ref[...] = lax.bitcast_convert_type((u0 << 16) | u1, jnp.float32)
    xr = x_ref[...]
    xw_ref[...] = jnp.concatenate(
        [xr[:, w * S_PER_W:(w + 1) * S_PER_W].reshape(IDX_PER_W)
         for w in range(WPB)])


_proj = pl.pallas_call(
    _proj_body,
    grid=(NBLK,),
    in_specs=[
        pl.BlockSpec((N_CLASSES, DIM), lambda i: (0, 0)),
        pl.BlockSpec((DIM, BLKN), lambda i: (0, i)),
        pl.BlockSpec((L, WPB * S_PER_W), lambda i: (0, i)),
    ],
    out_specs=[
        pl.BlockSpec((BLKN,), lambda i: (i,)),
        pl.BlockSpec((WPB * IDX_PER_W,), lambda i: (i,)),
    ],
    out_shape=[
        jax.ShapeDtypeStruct((NBLK * BLKN,), jnp.float32),
        jax.ShapeDtypeStruct((B * L,), jnp.int32),
    ],
)

_sc_mesh = plsc.VectorSubcoreMesh(core_axis_name="c", subcore_axis_name="s")


@functools.partial(
    pl.kernel,
    mesh=_sc_mesh,
    compiler_params=pltpu.CompilerParams(
        use_tc_tiling_on_sc=False, needs_layout_passes=False),
    out_type=jax.ShapeDtypeStruct((N_CLASSES * B,), jnp.float32),
    scratch_types=[
        pltpu.VMEM((IDX_PER_W,), jnp.int32),
        pltpu.VMEM((2, (L // 8) * S_PER_W), jnp.float32),
        pltpu.VMEM((N_CLASSES * S_PER_W,), jnp.float32),
        pltpu.VMEM((16,), jnp.float32),
        pltpu.SemaphoreType.DMA,
    ],
)
def _sc_pool(xw_hbm, pf_hbm, b_hbm, out_hbm, idx_v, g_v, out_v, b_v, sem0):
    wid = lax.axis_index("s") * NC + lax.axis_index("c")
    base = wid * S_PER_W
    pltpu.sync_copy(xw_hbm.at[pl.ds(wid * IDX_PER_W, IDX_PER_W)], idx_v)
    pltpu.sync_copy(b_hbm, b_v)

    NCHUNK = 8
    TCHUNK = L // NCHUNK
    CROWS = TCHUNK * S_PER_W

    def fire(c):
        buf = g_v.at[c % 2]

        def fire_t(tl, carry):
            t = c * TCHUNK + tl
            pltpu.async_copy(
                pf_hbm.at[idx_v.at[pl.ds(t * S_PER_W, S_PER_W)]],
                buf.at[pl.ds(tl * S_PER_W, S_PER_W)], sem0)
            return carry
        lax.fori_loop(0, TCHUNK, fire_t, 0)

    def drain(c):
        pltpu.make_async_copy(
            pf_hbm.at[pl.ds(0, CROWS)], g_v.at[c % 2], sem0).wait()

    zero = jnp.zeros((16,), jnp.float32)
    ngrp = S_PER_W // 16
    himask = jnp.full((16,), 0xFFFF0000, jnp.uint32)

    def accum(c, accs):
        buf = g_v.at[c % 2]

        def tok_body(tl, accs):
            goff = tl * S_PER_W
            new = list(accs)
            for j in range(ngrp):
                w = buf[pl.ds(goff + j * 16, 16)]
                u = plsc.bitcast(w, jnp.uint32)
                c0 = plsc.bitcast(u & himask, jnp.float32)
                c1 = plsc.bitcast(u << 16, jnp.float32)
                new[j] = new[j] + c0
                new[ngrp + j] = new[ngrp + j] + c1
            return tuple(new)
        return lax.fori_loop(0, TCHUNK, tok_body, accs)

    fire(0)
    accs = (zero,) * (2 * ngrp)
    for c in range(NCHUNK):
        if c + 1 < NCHUNK:
            fire(c + 1)
        drain(c)
        accs = accum(c, accs)

    bvec = b_v[pl.ds(0, 16)]
    b0 = bvec[0]
    b1 = bvec[1]
    for j in range(ngrp):
        out_v[pl.ds(j * 16, 16)] = accs[j] * INV_L + b0
        out_v[pl.ds(S_PER_W + j * 16, 16)] = accs[ngrp + j] * INV_L + b1
    pltpu.sync_copy(out_v.at[pl.ds(0, S_PER_W)],
                    out_hbm.at[pl.ds(base, S_PER_W)])
    pltpu.sync_copy(out_v.at[pl.ds(S_PER_W, S_PER_W)],
                    out_hbm.at[pl.ds(B + base, S_PER_W)])


@jax.jit
def kernel(x, table, W, b):
    tableT = table.T
    xT = x.T.astype(jnp.int32)
    pf, xw = _proj(W, tableT, xT)
    bpad = jnp.zeros((16,), jnp.float32).at[:N_CLASSES].set(b)
    out_t = _sc_pool(xw, pf, bpad)
    return out_t.reshape(N_CLASSES, B).T
